# Initial kernel scaffold; baseline (speedup 1.0000x reference)
import jax
import jax.numpy as jnp
from jax.experimental import pallas as pl


def kernel(atomic_num, degree, formal_charge, hybridization, W_atomic_num, W_degree, W_formal_charge, W_hybridization):
    raise NotImplementedError("write your pallas kernel here")


# speedup vs baseline: 2.8290x; 2.8290x over previous
import functools

import jax
import jax.numpy as jnp
from jax import lax
from jax.experimental import pallas as pl
from jax.experimental.pallas import tpu as pltpu
from jax.experimental.pallas import tpu_sc as plsc

N = 1_600_000
D = 32
L = 16
NC, NS = 2, 16
NW = NC * NS
NROW = N // 128
ROWS_W = 390
TAIL = NROW - ROWS_W * NW
R = 5
C = ROWS_W // R
G = R * 128 // L
OUTW = R * 128 * D

_mesh = plsc.VectorSubcoreMesh(core_axis_name="c", subcore_axis_name="s")


@functools.partial(
    pl.kernel,
    out_type=jax.ShapeDtypeStruct((N * D,), jnp.float32),
    mesh=_mesh,
    compiler_params=pltpu.CompilerParams(needs_layout_passes=False,
                                         use_tc_tiling_on_sc=False),
    scratch_types=[
        pltpu.VMEM((124 * D,), jnp.float32),
        pltpu.VMEM((16 * D,), jnp.float32),
        pltpu.VMEM((24 * D,), jnp.float32),
        pltpu.VMEM((16 * D,), jnp.float32),
        pltpu.VMEM((2, 4, R, 128), jnp.int32),
        pltpu.VMEM((2, OUTW), jnp.float32),
        pltpu.SemaphoreType.DMA,
        pltpu.SemaphoreType.DMA,
        pltpu.SemaphoreType.DMA,
        pltpu.SemaphoreType.DMA,
    ],
)
def _emb_kernel(an_hbm, de_hbm, fc_hbm, hy_hbm,
                wa_hbm, wd_hbm, wf_hbm, wh_hbm,
                out_hbm,
                wa_v, wd_v, wf_v, wh_v, idx_v, out_v,
                sem_in0, sem_in1, sem_out0, sem_out1):
    wid = lax.axis_index("s") * NC + lax.axis_index("c")
    base = wid * ROWS_W

    pltpu.sync_copy(wa_hbm, wa_v)
    pltpu.sync_copy(wd_hbm, wd_v)
    pltpu.sync_copy(wf_hbm, wf_v)
    pltpu.sync_copy(wh_hbm, wh_v)

    idx_hbms = (an_hbm, de_hbm, fc_hbm, hy_hbm)
    sems_in = (sem_in0, sem_in1)
    sems_out = (sem_out0, sem_out1)

    def issue_in(k, s):
        r0 = base + k * R
        for f in range(4):
            pltpu.async_copy(idx_hbms[f].at[pl.ds(r0, R)], idx_v.at[s, f],
                             sems_in[s])

    def wait_in(s):
        for f in range(4):
            pltpu.make_async_copy(idx_hbms[f].at[pl.ds(0, R)], idx_v.at[s, f],
                                  sems_in[s]).wait()

    def issue_out(k, s):
        w0 = (base + k * R) * 128 * D
        pltpu.async_copy(out_v.at[s], out_hbm.at[pl.ds(w0, OUTW)], sems_out[s])

    def wait_out(s):
        pltpu.make_async_copy(out_v.at[s], out_hbm.at[pl.ds(0, OUTW)],
                              sems_out[s]).wait()

    iota32 = lax.broadcasted_iota(jnp.int32, (L,), 0) * D

    def compute(s):
        out_s = out_v.at[s]

        def gbody(g, carry):
            r = g >> 3
            sl = pl.ds((g & 7) * L, L)
            a = idx_v[s, 0, r, sl] * D
            d = idx_v[s, 1, r, sl] * D
            f = idx_v[s, 2, r, sl] * D
            h = idx_v[s, 3, r, sl] * D
            sbase = iota32 + g * (L * D)
            for c in range(D):
                v = (plsc.load_gather(wa_v, [a + c])
                     + plsc.load_gather(wd_v, [d + c])
                     + plsc.load_gather(wf_v, [f + c])
                     + plsc.load_gather(wh_v, [h + c]))
                plsc.store_scatter(out_s, [sbase + c], v)
            return carry

        lax.fori_loop(0, G, gbody, 0)

    issue_in(0, 0)
    issue_in(1, 1)

    def pair(i, carry):
        for s in (0, 1):
            k = 2 * i + s
            wait_in(s)

            @pl.when(i > 0)
            def _():
                wait_out(s)

            compute(s)
            issue_out(k, s)

            @pl.when(i < (C // 2) - 1)
            def _():
                issue_in(k + 2, s)

        return carry

    lax.fori_loop(0, C // 2, pair, 0)
    wait_out(0)
    wait_out(1)

    @pl.when(wid < TAIL)
    def _():
        rt = NW * ROWS_W + wid
        for f in range(4):
            pltpu.async_copy(idx_hbms[f].at[pl.ds(rt, 1)],
                             idx_v.at[0, f, pl.ds(0, 1)], sem_in0)
        for f in range(4):
            pltpu.make_async_copy(idx_hbms[f].at[pl.ds(rt, 1)],
                                  idx_v.at[0, f, pl.ds(0, 1)], sem_in0).wait()

        out_s = out_v.at[0]

        def tbody(g, carry):
            sl = pl.ds(g * L, L)
            a = idx_v[0, 0, 0, sl] * D
            d = idx_v[0, 1, 0, sl] * D
            f = idx_v[0, 2, 0, sl] * D
            h = idx_v[0, 3, 0, sl] * D
            sbase = iota32 + g * (L * D)
            for c in range(D):
                v = (plsc.load_gather(wa_v, [a + c])
                     + plsc.load_gather(wd_v, [d + c])
                     + plsc.load_gather(wf_v, [f + c])
                     + plsc.load_gather(wh_v, [h + c]))
                plsc.store_scatter(out_s, [sbase + c], v)
            return carry

        lax.fori_loop(0, 128 // L, tbody, 0)
        tw = 128 * D
        pltpu.async_copy(out_v.at[0, pl.ds(0, tw)],
                         out_hbm.at[pl.ds(rt * tw, tw)], sem_out0)
        pltpu.make_async_copy(out_v.at[0, pl.ds(0, tw)],
                              out_hbm.at[pl.ds(rt * tw, tw)], sem_out0).wait()


def kernel(atomic_num, degree, formal_charge, hybridization,
           W_atomic_num, W_degree, W_formal_charge, W_hybridization):
    idx2d = [x.reshape(NROW, 128) for x in
             (atomic_num, degree, formal_charge, hybridization)]
    wa = W_atomic_num.reshape(-1)
    wd = W_degree.reshape(-1)
    wf = jnp.pad(W_formal_charge, ((0, 3), (0, 0))).reshape(-1)
    wh = jnp.pad(W_hybridization, ((0, 2), (0, 0))).reshape(-1)
    out = _emb_kernel(*idx2d, wa, wd, wf, wh)
    return out.reshape(N, D)


# speedup vs baseline: 13.0253x; 4.6041x over previous
import functools

import jax
import jax.numpy as jnp
from jax import lax
from jax.experimental import pallas as pl
from jax.experimental.pallas import tpu as pltpu
from jax.experimental.pallas import tpu_sc as plsc

N = 1_600_000
D = 32
L = 16
NC, NS = 2, 16
NW = NC * NS
EW = 49920
TAILW = (N - EW * NW) // 128
BE = 640
C = EW // BE
G = BE // L
OUTW = BE * D
NCOMB = 11 * 16 * 9

_mesh = plsc.VectorSubcoreMesh(core_axis_name="c", subcore_axis_name="s")


@functools.partial(
    pl.kernel,
    out_type=jax.ShapeDtypeStruct((N * D,), jnp.float32),
    mesh=_mesh,
    compiler_params=pltpu.CompilerParams(needs_layout_passes=False,
                                         use_tc_tiling_on_sc=False),
    scratch_types=[
        pltpu.VMEM((124 * D,), jnp.float32),
        pltpu.VMEM((16 * D,), jnp.float32),
        pltpu.VMEM((24 * D,), jnp.float32),
        pltpu.VMEM((16 * D,), jnp.float32),
        pltpu.VMEM((NCOMB * D,), jnp.float32),
        pltpu.VMEM((2, 4, BE), jnp.int32),
        pltpu.VMEM((2, OUTW), jnp.float32),
        pltpu.SemaphoreType.DMA,
        pltpu.SemaphoreType.DMA,
        pltpu.SemaphoreType.DMA,
        pltpu.SemaphoreType.DMA,
    ],
)
def _emb_kernel(an_hbm, de_hbm, fc_hbm, hy_hbm,
                wa_hbm, wd_hbm, wf_hbm, wh_hbm,
                out_hbm,
                wa_v, wd_v, wf_v, wh_v, wc_v, idx_v, out_v,
                sem_in0, sem_in1, sem_out0, sem_out1):
    wid = lax.axis_index("s") * NC + lax.axis_index("c")
    base_e = wid * EW

    pltpu.sync_copy(wa_hbm, wa_v)
    pltpu.sync_copy(wd_hbm, wd_v)
    pltpu.sync_copy(wf_hbm, wf_v)
    pltpu.sync_copy(wh_hbm, wh_v)

    idx_hbms = (an_hbm, de_hbm, fc_hbm, hy_hbm)
    sems_in = (sem_in0, sem_in1)
    sems_out = (sem_out0, sem_out1)

    def issue_in(k, s):
        e0 = base_e + k * BE
        for f in range(4):
            pltpu.async_copy(idx_hbms[f].at[pl.ds(e0, BE)], idx_v.at[s, f],
                             sems_in[s])

    def wait_in(s):
        for f in range(4):
            pltpu.make_async_copy(idx_hbms[f].at[pl.ds(0, BE)],
                                  idx_v.at[s, f], sems_in[s]).wait()

    def issue_out(k, s):
        w0 = (base_e + k * BE) * D
        pltpu.async_copy(out_v.at[s], out_hbm.at[pl.ds(w0, OUTW)], sems_out[s])

    def wait_out(s):
        pltpu.make_async_copy(out_v.at[s], out_hbm.at[pl.ds(0, OUTW)],
                              sems_out[s]).wait()

    def build_ij(ij, carry):
        i = ij // 16
        j = ij - i * 16
        dl = wd_v[pl.ds(i * D, L)]
        dh = wd_v[pl.ds(i * D + L, L)]
        fl = wf_v[pl.ds(j * D, L)]
        fh = wf_v[pl.ds(j * D + L, L)]
        sl_ = dl + fl
        sh_ = dh + fh
        r0 = ij * 9 * D
        for k in range(9):
            wc_v[pl.ds(r0 + k * D, L)] = sl_ + wh_v[pl.ds(k * D, L)]
            wc_v[pl.ds(r0 + k * D + L, L)] = sh_ + wh_v[pl.ds(k * D + L, L)]
        return carry

    lax.fori_loop(0, 11 * 16, build_ij, 0)

    io0 = lax.broadcasted_iota(jnp.int32, (L,), 0)

    def compute(s):
        out_s = out_v.at[s]

        def gbody(g, carry):
            sl = pl.ds(g * L, L)
            a32 = idx_v[s, 0, sl] * D
            d = idx_v[s, 1, sl]
            f = idx_v[s, 2, sl]
            h = idx_v[s, 3, sl]
            c32 = ((d * 16 + f) * 9 + h) * D
            ob = g * (L * D)
            for e in range(L):
                ee = jnp.full((L,), e, jnp.int32)
                ba = jnp.take(a32, ee) + io0
                bc = jnp.take(c32, ee) + io0
                lo = plsc.load_gather(wa_v, [ba]) + plsc.load_gather(wc_v, [bc])
                hi = (plsc.load_gather(wa_v, [ba + L])
                      + plsc.load_gather(wc_v, [bc + L]))
                out_s[pl.ds(ob + e * D, L)] = lo
                out_s[pl.ds(ob + e * D + L, L)] = hi
            return carry

        lax.fori_loop(0, G, gbody, 0)

    issue_in(0, 0)
    issue_in(1, 1)

    def pair(i, carry):
        for s in (0, 1):
            k = 2 * i + s
            wait_in(s)

            @pl.when(i > 0)
            def _():
                wait_out(s)

            compute(s)
            issue_out(k, s)

            @pl.when(i < (C // 2) - 1)
            def _():
                issue_in(k + 2, s)

        return carry

    lax.fori_loop(0, C // 2, pair, 0)
    wait_out(0)
    wait_out(1)

    @pl.when(wid < TAILW)
    def _():
        et = NW * EW + wid * 128
        for f in range(4):
            pltpu.async_copy(idx_hbms[f].at[pl.ds(et, 128)],
                             idx_v.at[0, f, pl.ds(0, 128)], sem_in0)
        for f in range(4):
            pltpu.make_async_copy(idx_hbms[f].at[pl.ds(et, 128)],
                                  idx_v.at[0, f, pl.ds(0, 128)], sem_in0).wait()

        out_s = out_v.at[0]

        def tbody(g, carry):
            sl = pl.ds(g * L, L)
            a32 = idx_v[0, 0, sl] * D
            d = idx_v[0, 1, sl]
            f = idx_v[0, 2, sl]
            h = idx_v[0, 3, sl]
            c32 = ((d * 16 + f) * 9 + h) * D
            ob = g * (L * D)
            for e in range(L):
                ee = jnp.full((L,), e, jnp.int32)
                ba = jnp.take(a32, ee) + io0
                bc = jnp.take(c32, ee) + io0
                lo = plsc.load_gather(wa_v, [ba]) + plsc.load_gather(wc_v, [bc])
                hi = (plsc.load_gather(wa_v, [ba + L])
                      + plsc.load_gather(wc_v, [bc + L]))
                out_s[pl.ds(ob + e * D, L)] = lo
                out_s[pl.ds(ob + e * D + L, L)] = hi
            return carry

        lax.fori_loop(0, 128 // L, tbody, 0)
        tw = 128 * D
        pltpu.async_copy(out_v.at[0, pl.ds(0, tw)],
                         out_hbm.at[pl.ds(et * D, tw)], sem_out0)
        pltpu.make_async_copy(out_v.at[0, pl.ds(0, tw)],
                              out_hbm.at[pl.ds(et * D, tw)], sem_out0).wait()


def kernel(atomic_num, degree, formal_charge, hybridization,
           W_atomic_num, W_degree, W_formal_charge, W_hybridization):
    wa = W_atomic_num.reshape(-1)
    wd = W_degree.reshape(-1)
    wf = jnp.pad(W_formal_charge, ((0, 3), (0, 0))).reshape(-1)
    wh = jnp.pad(W_hybridization, ((0, 2), (0, 0))).reshape(-1)
    out = _emb_kernel(atomic_num, degree, formal_charge, hybridization,
                      wa, wd, wf, wh)
    return out.reshape(N, D)
